# Initial kernel scaffold; baseline (speedup 1.0000x reference)
import jax
import jax.numpy as jnp
from jax import lax
from jax.experimental import pallas as pl
from jax.experimental.pallas import tpu as pltpu
from jax.experimental.pallas import tpu_sc as plsc

N_GRAPHS = 1024
EMB_ATOMIC = 64
N_CHARGE = 32
EMB_CHARGE = 16
CONT_IN = 8
EMB_CONT = 32
TOTAL_DIM = EMB_ATOMIC + EMB_CHARGE + EMB_CONT
OUT_DIM = 128

NW = 32
SL = 128
STRIPS = 25
PER_W = SL * STRIPS
NPAD = NW * PER_W
BLK = 1024


def _prep_body(tc_ref, ec_ref, w2_ref, wp_ref, b2_ref,
               gg_ref, wf_ref, bias_ref):
    tc = tc_ref[...]
    oh = (tc == lax.broadcasted_iota(jnp.int32, (N_GRAPHS, N_CHARGE), 1))
    gg_ref[...] = jnp.dot(oh.astype(jnp.float32), ec_ref[...],
                          preferred_element_type=jnp.float32)
    wp = wp_ref[...]
    wf_ref[...] = jnp.concatenate(
        [wp[:EMB_ATOMIC + EMB_CHARGE],
         jnp.dot(w2_ref[...], wp[EMB_ATOMIC + EMB_CHARGE:],
                 preferred_element_type=jnp.float32)], axis=0)
    bias_ref[...] = jnp.dot(b2_ref[...], wp[EMB_ATOMIC + EMB_CHARGE:],
                            preferred_element_type=jnp.float32)


_prep = pl.pallas_call(
    _prep_body,
    out_shape=[
        jax.ShapeDtypeStruct((N_GRAPHS, EMB_CHARGE), jnp.float32),
        jax.ShapeDtypeStruct((TOTAL_DIM, OUT_DIM), jnp.float32),
        jax.ShapeDtypeStruct((1, OUT_DIM), jnp.float32),
    ],
)


def _gather_body(at_hbm, b_hbm, table_hbm, gg_hbm, a_out, c_out,
                 at_v, b_v, a0, a1, c0, c1, sem0, sem1):
    cid = lax.axis_index("c")
    sid = lax.axis_index("s")
    wid = sid * 2 + cid
    base = wid * PER_W
    pltpu.sync_copy(at_hbm.at[wid], at_v)
    pltpu.sync_copy(b_hbm.at[wid], b_v)
    abufs, cbufs, sems = (a0, a1), (c0, c1), (sem0, sem1)
    pending = {}

    def start(j):
        p = j & 1
        d1 = pltpu.async_copy(table_hbm.at[at_v.at[j]], abufs[p], sems[p])
        d2 = pltpu.async_copy(gg_hbm.at[b_v.at[j]], cbufs[p], sems[p])
        pending[j] = (d1, d2)

    start(0)
    for j in range(STRIPS):
        if j + 1 < STRIPS:
            start(j + 1)
        d1, d2 = pending.pop(j)
        d1.wait()
        d2.wait()
        p = j & 1
        pltpu.sync_copy(abufs[p], a_out.at[pl.ds(base + j * SL, SL)])
        pltpu.sync_copy(cbufs[p], c_out.at[pl.ds(base + j * SL, SL)])


_gather = pl.kernel(
    _gather_body,
    out_type=(
        jax.ShapeDtypeStruct((NPAD, EMB_ATOMIC), jnp.float32),
        jax.ShapeDtypeStruct((NPAD, EMB_CHARGE), jnp.float32),
    ),
    mesh=plsc.VectorSubcoreMesh(core_axis_name="c", subcore_axis_name="s"),
    scratch_types=[
        pltpu.VMEM((STRIPS, SL), jnp.int32),
        pltpu.VMEM((STRIPS, SL), jnp.int32),
        pltpu.VMEM((SL, EMB_ATOMIC), jnp.float32),
        pltpu.VMEM((SL, EMB_ATOMIC), jnp.float32),
        pltpu.VMEM((SL, EMB_CHARGE), jnp.float32),
        pltpu.VMEM((SL, EMB_CHARGE), jnp.float32),
        pltpu.SemaphoreType.DMA,
        pltpu.SemaphoreType.DMA,
    ],
)


def _dense_body(a_ref, c_ref, ef_ref, wf_ref, bias_ref, w1_ref, b1_ref,
                out_ref):
    h = jnp.dot(ef_ref[...], w1_ref[...],
                preferred_element_type=jnp.float32) + b1_ref[...]
    h = h * lax.logistic(h)
    x = jnp.concatenate([a_ref[...], c_ref[...], h], axis=1)
    y = jnp.dot(x, wf_ref[...], preferred_element_type=jnp.float32)
    y = y + bias_ref[...]
    out_ref[...] = y * lax.logistic(y)


_dense = pl.pallas_call(
    _dense_body,
    grid=(NPAD // BLK,),
    in_specs=[
        pl.BlockSpec((BLK, EMB_ATOMIC), lambda b: (b, 0)),
        pl.BlockSpec((BLK, EMB_CHARGE), lambda b: (b, 0)),
        pl.BlockSpec((BLK, CONT_IN), lambda b: (b, 0)),
        pl.BlockSpec((TOTAL_DIM, OUT_DIM), lambda b: (0, 0)),
        pl.BlockSpec((1, OUT_DIM), lambda b: (0, 0)),
        pl.BlockSpec((CONT_IN, EMB_CONT), lambda b: (0, 0)),
        pl.BlockSpec((1, EMB_CONT), lambda b: (0, 0)),
    ],
    out_specs=pl.BlockSpec((BLK, OUT_DIM), lambda b: (b, 0)),
    out_shape=jax.ShapeDtypeStruct((NPAD, OUT_DIM), jnp.float32),
)


def kernel(batch, atomic_type, total_charge, external_field,
           emb_atomic, emb_charge, W1, b1, W2, b2, Wp):
    n = batch.shape[0]
    pad = NPAD - n
    at = jnp.pad(atomic_type.astype(jnp.int32).reshape(-1), (0, pad))
    bt = jnp.pad(batch.astype(jnp.int32), (0, pad))
    ef = jnp.pad(external_field, ((0, pad), (0, 0)))

    gg, wf, bias = _prep(total_charge.astype(jnp.int32).reshape(-1, 1),
                         emb_charge, W2, Wp, b2.reshape(1, -1))
    a_rows, c_rows = _gather(at.reshape(NW, STRIPS, SL),
                             bt.reshape(NW, STRIPS, SL), emb_atomic, gg)
    y = _dense(a_rows, c_rows, ef, wf, bias, W1, b1.reshape(1, -1))
    return y[:n]


# speedup vs baseline: 2.1375x; 2.1375x over previous
import functools

import jax
import jax.numpy as jnp
from jax import lax
from jax.experimental import pallas as pl
from jax.experimental.pallas import tpu as pltpu
from jax.experimental.pallas import tpu_sc as plsc

N_GRAPHS = 1024
EMB_ATOMIC = 64
N_CHARGE = 32
EMB_CHARGE = 16
CONT_IN = 8
EMB_CONT = 32
TOTAL_DIM = EMB_ATOMIC + EMB_CHARGE + EMB_CONT
OUT_DIM = 128

NW = 32
SL = 128
STRIPS = 25
PER_W = SL * STRIPS
NPAD = NW * PER_W
BLK = 1024


def _prep_body(tc_ref, ec_ref, w2_ref, wp_ref, b2_ref,
               gg_ref, wf_ref, bias_ref):
    tc = tc_ref[...]
    oh = (tc == lax.broadcasted_iota(jnp.int32, (N_GRAPHS, N_CHARGE), 1))
    gg_ref[...] = jnp.dot(oh.astype(jnp.float32), ec_ref[...],
                          preferred_element_type=jnp.float32)
    wp = wp_ref[...]
    wf_ref[...] = jnp.concatenate(
        [wp[:EMB_ATOMIC + EMB_CHARGE],
         jnp.dot(w2_ref[...], wp[EMB_ATOMIC + EMB_CHARGE:],
                 preferred_element_type=jnp.float32)], axis=0)
    bias_ref[...] = jnp.dot(b2_ref[...], wp[EMB_ATOMIC + EMB_CHARGE:],
                            preferred_element_type=jnp.float32)


_prep = pl.pallas_call(
    _prep_body,
    out_shape=[
        jax.ShapeDtypeStruct((N_GRAPHS, EMB_CHARGE), jnp.float32),
        jax.ShapeDtypeStruct((TOTAL_DIM, OUT_DIM), jnp.float32),
        jax.ShapeDtypeStruct((1, OUT_DIM), jnp.float32),
    ],
)


def _gather_body(at_hbm, b_hbm, table_hbm, gg_hbm, a_out, c_out,
                 at_v, b_v, a0, a1, c0, c1, sem0, sem1):
    cid = lax.axis_index("c")
    sid = lax.axis_index("s")
    wid = sid * 2 + cid
    base = wid * PER_W
    pltpu.sync_copy(at_hbm.at[wid], at_v)
    pltpu.sync_copy(b_hbm.at[wid], b_v)
    abufs, cbufs, sems = (a0, a1), (c0, c1), (sem0, sem1)
    pending = {}

    def start(j):
        p = j & 1
        d1 = pltpu.async_copy(table_hbm.at[at_v.at[j]], abufs[p], sems[p])
        d2 = pltpu.async_copy(gg_hbm.at[b_v.at[j]], cbufs[p], sems[p])
        pending[j] = (d1, d2)

    start(0)
    for j in range(STRIPS):
        if j + 1 < STRIPS:
            start(j + 1)
        d1, d2 = pending.pop(j)
        d1.wait()
        d2.wait()
        p = j & 1
        pltpu.sync_copy(abufs[p], a_out.at[pl.ds(base + j * SL, SL)])
        pltpu.sync_copy(cbufs[p], c_out.at[pl.ds(base + j * SL, SL)])


@functools.lru_cache(maxsize=None)
def _make_gather():
  return pl.kernel(
    _gather_body,
    out_type=(
        jax.ShapeDtypeStruct((NPAD, EMB_ATOMIC), jnp.float32),
        jax.ShapeDtypeStruct((NPAD, EMB_CHARGE), jnp.float32),
    ),
    mesh=plsc.VectorSubcoreMesh(core_axis_name="c", subcore_axis_name="s",
                                num_cores=2, num_subcores=16),
    compiler_params=pltpu.CompilerParams(use_tc_tiling_on_sc=False),
    scratch_types=[
        pltpu.VMEM((STRIPS, SL), jnp.int32),
        pltpu.VMEM((STRIPS, SL), jnp.int32),
        pltpu.VMEM((SL, EMB_ATOMIC), jnp.float32),
        pltpu.VMEM((SL, EMB_ATOMIC), jnp.float32),
        pltpu.VMEM((SL, EMB_CHARGE), jnp.float32),
        pltpu.VMEM((SL, EMB_CHARGE), jnp.float32),
        pltpu.SemaphoreType.DMA,
        pltpu.SemaphoreType.DMA,
    ],
  )


def _dense_body(a_ref, c_ref, ef_ref, wf_ref, bias_ref, w1_ref, b1_ref,
                out_ref):
    h = jnp.dot(ef_ref[...], w1_ref[...],
                preferred_element_type=jnp.float32) + b1_ref[...]
    h = h * lax.logistic(h)
    x = jnp.concatenate([a_ref[...], c_ref[...], h], axis=1)
    y = jnp.dot(x, wf_ref[...], preferred_element_type=jnp.float32)
    y = y + bias_ref[...]
    out_ref[...] = y * lax.logistic(y)


_dense = pl.pallas_call(
    _dense_body,
    grid=(NPAD // BLK,),
    in_specs=[
        pl.BlockSpec((BLK, EMB_ATOMIC), lambda b: (b, 0)),
        pl.BlockSpec((BLK, EMB_CHARGE), lambda b: (b, 0)),
        pl.BlockSpec((BLK, CONT_IN), lambda b: (b, 0)),
        pl.BlockSpec((TOTAL_DIM, OUT_DIM), lambda b: (0, 0)),
        pl.BlockSpec((1, OUT_DIM), lambda b: (0, 0)),
        pl.BlockSpec((CONT_IN, EMB_CONT), lambda b: (0, 0)),
        pl.BlockSpec((1, EMB_CONT), lambda b: (0, 0)),
    ],
    out_specs=pl.BlockSpec((BLK, OUT_DIM), lambda b: (b, 0)),
    out_shape=jax.ShapeDtypeStruct((NPAD, OUT_DIM), jnp.float32),
)


def kernel(batch, atomic_type, total_charge, external_field,
           emb_atomic, emb_charge, W1, b1, W2, b2, Wp):
    n = batch.shape[0]
    pad = NPAD - n
    at = jnp.pad(atomic_type.astype(jnp.int32).reshape(-1), (0, pad))
    bt = jnp.pad(batch.astype(jnp.int32), (0, pad))
    ef = jnp.pad(external_field, ((0, pad), (0, 0)))

    gg, wf, bias = _prep(total_charge.astype(jnp.int32).reshape(-1, 1),
                         emb_charge, W2, Wp, b2.reshape(1, -1))
    a_rows, c_rows = _make_gather()(at.reshape(NW, STRIPS, SL),
                                    bt.reshape(NW, STRIPS, SL), emb_atomic, gg)
    y = _dense(a_rows, c_rows, ef, wf, bias, W1, b1.reshape(1, -1))
    return y[:n]


# speedup vs baseline: 2.5294x; 1.1833x over previous
import functools

import jax
import jax.numpy as jnp
from jax import lax
from jax.experimental import pallas as pl
from jax.experimental.pallas import tpu as pltpu
from jax.experimental.pallas import tpu_sc as plsc

N_GRAPHS = 1024
EMB_ATOMIC = 64
N_CHARGE = 32
EMB_CHARGE = 16
CONT_IN = 8
EMB_CONT = 32
TOTAL_DIM = EMB_ATOMIC + EMB_CHARGE + EMB_CONT
OUT_DIM = 128

NW = 32
SL = 128
STRIPS = 25
SS_DMAS = 5
SS = SL * SS_DMAS
SUPERS = STRIPS // SS_DMAS
PER_W = SL * STRIPS
NPAD = NW * PER_W
BLK = 1024


def _prep_body(tc_ref, ec_ref, w2_ref, wp_ref, b2_ref,
               gg_ref, wf_ref, bias_ref):
    tc = tc_ref[...]
    oh = (tc == lax.broadcasted_iota(jnp.int32, (N_GRAPHS, N_CHARGE), 1))
    gg_ref[...] = jnp.dot(oh.astype(jnp.float32), ec_ref[...],
                          preferred_element_type=jnp.float32)
    wp = wp_ref[...]
    wf_ref[...] = jnp.concatenate(
        [wp[:EMB_ATOMIC + EMB_CHARGE],
         jnp.dot(w2_ref[...], wp[EMB_ATOMIC + EMB_CHARGE:],
                 preferred_element_type=jnp.float32)], axis=0)
    bias_ref[...] = jnp.dot(b2_ref[...], wp[EMB_ATOMIC + EMB_CHARGE:],
                            preferred_element_type=jnp.float32)


_prep = pl.pallas_call(
    _prep_body,
    out_shape=[
        jax.ShapeDtypeStruct((N_GRAPHS, EMB_CHARGE), jnp.float32),
        jax.ShapeDtypeStruct((TOTAL_DIM, OUT_DIM), jnp.float32),
        jax.ShapeDtypeStruct((1, OUT_DIM), jnp.float32),
    ],
)


def _gather_body(at_hbm, b_hbm, table_hbm, gg_hbm, a_out, c_out,
                 at_v, b_v, a0, a1, c0, c1, sem0, sem1, osem0, osem1):
    cid = lax.axis_index("c")
    sid = lax.axis_index("s")
    wid = sid * 2 + cid
    base = wid * PER_W
    pltpu.sync_copy(at_hbm.at[wid], at_v)
    pltpu.sync_copy(b_hbm.at[wid], b_v)
    abufs, cbufs = (a0, a1), (c0, c1)
    sems, osems = (sem0, sem1), (osem0, osem1)
    pending = {}
    draining = {}

    def start(j):
        p = j & 1
        ds = []
        for k in range(SS_DMAS):
            r = j * SS_DMAS + k
            ds.append(pltpu.async_copy(
                table_hbm.at[at_v.at[r]], abufs[p].at[pl.ds(k * SL, SL)],
                sems[p]))
            ds.append(pltpu.async_copy(
                gg_hbm.at[b_v.at[r]], cbufs[p].at[pl.ds(k * SL, SL)],
                sems[p]))
        pending[j] = ds

    def drain(j):
        p = j & 1
        draining[j] = (
            pltpu.async_copy(abufs[p], a_out.at[pl.ds(base + j * SS, SS)],
                             osems[p]),
            pltpu.async_copy(cbufs[p], c_out.at[pl.ds(base + j * SS, SS)],
                             osems[p]),
        )

    start(0)
    for j in range(SUPERS):
        if j >= 1:
            for d in draining.pop(j - 1):
                d.wait()
        if j + 1 < SUPERS:
            start(j + 1)
        for d in pending.pop(j):
            d.wait()
        drain(j)
    for d in draining.pop(SUPERS - 1):
        d.wait()


@functools.lru_cache(maxsize=None)
def _make_gather():
  return pl.kernel(
    _gather_body,
    out_type=(
        jax.ShapeDtypeStruct((NPAD, EMB_ATOMIC), jnp.float32),
        jax.ShapeDtypeStruct((NPAD, EMB_CHARGE), jnp.float32),
    ),
    mesh=plsc.VectorSubcoreMesh(core_axis_name="c", subcore_axis_name="s",
                                num_cores=2, num_subcores=16),
    compiler_params=pltpu.CompilerParams(use_tc_tiling_on_sc=False),
    scratch_types=[
        pltpu.VMEM((STRIPS, SL), jnp.int32),
        pltpu.VMEM((STRIPS, SL), jnp.int32),
        pltpu.VMEM((SS, EMB_ATOMIC), jnp.float32),
        pltpu.VMEM((SS, EMB_ATOMIC), jnp.float32),
        pltpu.VMEM((SS, EMB_CHARGE), jnp.float32),
        pltpu.VMEM((SS, EMB_CHARGE), jnp.float32),
        pltpu.SemaphoreType.DMA,
        pltpu.SemaphoreType.DMA,
        pltpu.SemaphoreType.DMA,
        pltpu.SemaphoreType.DMA,
    ],
  )


def _dense_body(a_ref, c_ref, ef_ref, wf_ref, bias_ref, w1_ref, b1_ref,
                out_ref):
    h = jnp.dot(ef_ref[...], w1_ref[...],
                preferred_element_type=jnp.float32) + b1_ref[...]
    h = h * lax.logistic(h)
    x = jnp.concatenate([a_ref[...], c_ref[...], h], axis=1)
    y = jnp.dot(x, wf_ref[...], preferred_element_type=jnp.float32)
    y = y + bias_ref[...]
    out_ref[...] = y * lax.logistic(y)


@functools.lru_cache(maxsize=None)
def _make_dense(n):
    return pl.pallas_call(
        _dense_body,
        grid=(-(-n // BLK),),
        in_specs=[
            pl.BlockSpec((BLK, EMB_ATOMIC), lambda b: (b, 0)),
            pl.BlockSpec((BLK, EMB_CHARGE), lambda b: (b, 0)),
            pl.BlockSpec((BLK, CONT_IN), lambda b: (b, 0)),
            pl.BlockSpec((TOTAL_DIM, OUT_DIM), lambda b: (0, 0)),
            pl.BlockSpec((1, OUT_DIM), lambda b: (0, 0)),
            pl.BlockSpec((CONT_IN, EMB_CONT), lambda b: (0, 0)),
            pl.BlockSpec((1, EMB_CONT), lambda b: (0, 0)),
        ],
        out_specs=pl.BlockSpec((BLK, OUT_DIM), lambda b: (b, 0)),
        out_shape=jax.ShapeDtypeStruct((n, OUT_DIM), jnp.float32),
    )


def kernel(batch, atomic_type, total_charge, external_field,
           emb_atomic, emb_charge, W1, b1, W2, b2, Wp):
    n = batch.shape[0]
    pad = NPAD - n
    at = jnp.pad(atomic_type.astype(jnp.int32).reshape(-1), (0, pad))
    bt = jnp.pad(batch.astype(jnp.int32), (0, pad))

    gg, wf, bias = _prep(total_charge.astype(jnp.int32).reshape(-1, 1),
                         emb_charge, W2, Wp, b2.reshape(1, -1))
    a_rows, c_rows = _make_gather()(at.reshape(NW, STRIPS, SL),
                                    bt.reshape(NW, STRIPS, SL), emb_atomic, gg)
    return _make_dense(n)(a_rows, c_rows, external_field, wf, bias,
                          W1, b1.reshape(1, -1))


# speedup vs baseline: 3.1906x; 1.2614x over previous
import functools

import jax
import jax.numpy as jnp
from jax import lax
from jax.experimental import pallas as pl
from jax.experimental.pallas import tpu as pltpu
from jax.experimental.pallas import tpu_sc as plsc

N_GRAPHS = 1024
EMB_ATOMIC = 64
N_CHARGE = 32
EMB_CHARGE = 16
CONT_IN = 8
EMB_CONT = 32
CAT_DIM = EMB_ATOMIC + EMB_CHARGE
TOTAL_DIM = CAT_DIM + EMB_CONT
OUT_DIM = 128

NW = 32
SL = 128
STRIPS = 25
SS_DMAS = 5
SS = SL * SS_DMAS
SUPERS = STRIPS // SS_DMAS
PER_W = SL * STRIPS
NPAD = NW * PER_W
BLK = 1024


def _prep_body(tc_ref, ec_ref, gg_ref):
    tc = tc_ref[...]
    oh = (tc == lax.broadcasted_iota(jnp.int32, (N_GRAPHS, N_CHARGE), 1))
    gg_ref[...] = jnp.dot(oh.astype(jnp.float32), ec_ref[...],
                          preferred_element_type=jnp.float32)


_prep = pl.pallas_call(
    _prep_body,
    out_shape=jax.ShapeDtypeStruct((N_GRAPHS, EMB_CHARGE), jnp.float32),
)


def _gather_body(at_hbm, b_hbm, table_hbm, gg_hbm, x_out,
                 at_v, b_v, a0, a1, c0, c1, sem0, sem1, osem0, osem1):
    cid = lax.axis_index("c")
    sid = lax.axis_index("s")
    wid = sid * 2 + cid
    base = wid * PER_W
    pltpu.sync_copy(at_hbm.at[wid], at_v)
    pltpu.sync_copy(b_hbm.at[wid], b_v)
    abufs, cbufs = (a0, a1), (c0, c1)
    sems, osems = (sem0, sem1), (osem0, osem1)
    pending = {}
    draining = {}

    def start(j):
        p = j & 1
        ds = []
        for k in range(SS_DMAS):
            r = j * SS_DMAS + k
            ds.append(pltpu.async_copy(
                table_hbm.at[at_v.at[r]], abufs[p].at[pl.ds(k * SL, SL)],
                sems[p]))
            ds.append(pltpu.async_copy(
                gg_hbm.at[b_v.at[r]], cbufs[p].at[pl.ds(k * SL, SL)],
                sems[p]))
        pending[j] = ds

    def drain(j):
        p = j & 1
        rows = pl.ds(base + j * SS, SS)
        draining[j] = (
            pltpu.async_copy(abufs[p], x_out.at[rows, pl.ds(0, EMB_ATOMIC)],
                             osems[p]),
            pltpu.async_copy(cbufs[p],
                             x_out.at[rows, pl.ds(EMB_ATOMIC, EMB_CHARGE)],
                             osems[p]),
        )

    start(0)
    for j in range(SUPERS):
        if j >= 1:
            for d in draining.pop(j - 1):
                d.wait()
        if j + 1 < SUPERS:
            start(j + 1)
        for d in pending.pop(j):
            d.wait()
        drain(j)
    for d in draining.pop(SUPERS - 1):
        d.wait()


@functools.lru_cache(maxsize=None)
def _make_gather():
  return pl.kernel(
    _gather_body,
    out_type=jax.ShapeDtypeStruct((NPAD, OUT_DIM), jnp.float32),
    mesh=plsc.VectorSubcoreMesh(core_axis_name="c", subcore_axis_name="s",
                                num_cores=2, num_subcores=16),
    compiler_params=pltpu.CompilerParams(use_tc_tiling_on_sc=False),
    scratch_types=[
        pltpu.VMEM((STRIPS, SL), jnp.int32),
        pltpu.VMEM((STRIPS, SL), jnp.int32),
        pltpu.VMEM((SS, EMB_ATOMIC), jnp.float32),
        pltpu.VMEM((SS, EMB_ATOMIC), jnp.float32),
        pltpu.VMEM((SS, EMB_CHARGE), jnp.float32),
        pltpu.VMEM((SS, EMB_CHARGE), jnp.float32),
        pltpu.SemaphoreType.DMA,
        pltpu.SemaphoreType.DMA,
        pltpu.SemaphoreType.DMA,
        pltpu.SemaphoreType.DMA,
    ],
  )


def _dense_body(x_ref, ef_ref, wp_ref, w1_ref, b1_ref, w2_ref, b2_ref,
                out_ref):
    h = jnp.dot(ef_ref[...], w1_ref[...],
                preferred_element_type=jnp.float32) + b1_ref[...]
    h = h * lax.logistic(h)
    wp = wp_ref[...]
    wh = jnp.dot(w2_ref[...], wp[CAT_DIM:], preferred_element_type=jnp.float32)
    bias = jnp.dot(b2_ref[...], wp[CAT_DIM:],
                   preferred_element_type=jnp.float32)
    y = jnp.dot(x_ref[...][:, :CAT_DIM], wp[:CAT_DIM],
                preferred_element_type=jnp.float32)
    y = y + jnp.dot(h, wh, preferred_element_type=jnp.float32) + bias
    out_ref[...] = y * lax.logistic(y)


@functools.lru_cache(maxsize=None)
def _make_dense(n):
    return pl.pallas_call(
        _dense_body,
        grid=(-(-n // BLK),),
        in_specs=[
            pl.BlockSpec((BLK, OUT_DIM), lambda b: (b, 0)),
            pl.BlockSpec((BLK, CONT_IN), lambda b: (b, 0)),
            pl.BlockSpec((TOTAL_DIM, OUT_DIM), lambda b: (0, 0)),
            pl.BlockSpec((CONT_IN, EMB_CONT), lambda b: (0, 0)),
            pl.BlockSpec((1, EMB_CONT), lambda b: (0, 0)),
            pl.BlockSpec((EMB_CONT, EMB_CONT), lambda b: (0, 0)),
            pl.BlockSpec((1, EMB_CONT), lambda b: (0, 0)),
        ],
        out_specs=pl.BlockSpec((BLK, OUT_DIM), lambda b: (b, 0)),
        out_shape=jax.ShapeDtypeStruct((n, OUT_DIM), jnp.float32),
    )


def kernel(batch, atomic_type, total_charge, external_field,
           emb_atomic, emb_charge, W1, b1, W2, b2, Wp):
    n = batch.shape[0]
    pad = NPAD - n
    at = jnp.pad(atomic_type.astype(jnp.int32).reshape(-1), (0, pad))
    bt = jnp.pad(batch.astype(jnp.int32), (0, pad))

    gg = _prep(total_charge.astype(jnp.int32).reshape(-1, 1), emb_charge)
    x = _make_gather()(at.reshape(NW, STRIPS, SL),
                       bt.reshape(NW, STRIPS, SL), emb_atomic, gg)
    return _make_dense(n)(x, external_field, Wp, W1, b1.reshape(1, -1),
                          W2, b2.reshape(1, -1))


# speedup vs baseline: 3.1955x; 1.0015x over previous
import functools

import jax
import jax.numpy as jnp
from jax import lax
from jax.experimental import pallas as pl
from jax.experimental.pallas import tpu as pltpu
from jax.experimental.pallas import tpu_sc as plsc

N_GRAPHS = 1024
EMB_ATOMIC = 64
N_CHARGE = 32
EMB_CHARGE = 16
CONT_IN = 8
EMB_CONT = 32
CAT_DIM = EMB_ATOMIC + EMB_CHARGE
TOTAL_DIM = CAT_DIM + EMB_CONT
OUT_DIM = 128

NW = 32
SL = 128
STRIPS = 25
SS_DMAS = 5
SS = SL * SS_DMAS
SUPERS = STRIPS // SS_DMAS
PER_W = SL * STRIPS
NPAD = NW * PER_W
BLK = 1024


def _prep_body(tc_ref, ec_ref, gg_ref):
    tc = tc_ref[...]
    oh = (tc == lax.broadcasted_iota(jnp.int32, (N_GRAPHS, N_CHARGE), 1))
    gg_ref[...] = jnp.dot(oh.astype(jnp.float32), ec_ref[...],
                          preferred_element_type=jnp.float32)


_prep = pl.pallas_call(
    _prep_body,
    out_shape=jax.ShapeDtypeStruct((N_GRAPHS, EMB_CHARGE), jnp.float32),
)


def _gather_body(at_hbm, b_hbm, table_hbm, gg_hbm, x_out,
                 at_v, b_v, a0, a1, c0, c1, sem0, sem1, osem0, osem1):
    cid = lax.axis_index("c")
    sid = lax.axis_index("s")
    wid = cid * 16 + sid
    base = wid * PER_W
    pltpu.sync_copy(at_hbm.at[pl.ds(wid * STRIPS, STRIPS)], at_v)
    pltpu.sync_copy(b_hbm.at[pl.ds(wid * STRIPS, STRIPS)], b_v)
    abufs, cbufs = (a0, a1), (c0, c1)
    sems, osems = (sem0, sem1), (osem0, osem1)
    pending = {}
    draining = {}

    def start(j):
        p = j & 1
        ds = []
        for k in range(SS_DMAS):
            r = j * SS_DMAS + k
            ds.append(pltpu.async_copy(
                table_hbm.at[at_v.at[r]], abufs[p].at[pl.ds(k * SL, SL)],
                sems[p]))
            ds.append(pltpu.async_copy(
                gg_hbm.at[b_v.at[r]], cbufs[p].at[pl.ds(k * SL, SL)],
                sems[p]))
        pending[j] = ds

    def drain(j):
        p = j & 1
        rows = pl.ds(base + j * SS, SS)
        draining[j] = (
            pltpu.async_copy(abufs[p], x_out.at[rows, pl.ds(0, EMB_ATOMIC)],
                             osems[p]),
            pltpu.async_copy(cbufs[p],
                             x_out.at[rows, pl.ds(EMB_ATOMIC, EMB_CHARGE)],
                             osems[p]),
        )

    start(0)
    for j in range(SUPERS):
        if j >= 1:
            for d in draining.pop(j - 1):
                d.wait()
        if j + 1 < SUPERS:
            start(j + 1)
        for d in pending.pop(j):
            d.wait()
        drain(j)
    for d in draining.pop(SUPERS - 1):
        d.wait()


@functools.lru_cache(maxsize=None)
def _make_gather():
  return pl.kernel(
    _gather_body,
    out_type=jax.ShapeDtypeStruct((NPAD, OUT_DIM), jnp.float32),
    mesh=plsc.VectorSubcoreMesh(core_axis_name="c", subcore_axis_name="s",
                                num_cores=2, num_subcores=16),
    compiler_params=pltpu.CompilerParams(use_tc_tiling_on_sc=False),
    scratch_types=[
        pltpu.VMEM((STRIPS, SL), jnp.int32),
        pltpu.VMEM((STRIPS, SL), jnp.int32),
        pltpu.VMEM((SS, EMB_ATOMIC), jnp.float32),
        pltpu.VMEM((SS, EMB_ATOMIC), jnp.float32),
        pltpu.VMEM((SS, EMB_CHARGE), jnp.float32),
        pltpu.VMEM((SS, EMB_CHARGE), jnp.float32),
        pltpu.SemaphoreType.DMA,
        pltpu.SemaphoreType.DMA,
        pltpu.SemaphoreType.DMA,
        pltpu.SemaphoreType.DMA,
    ],
  )


def _dense_body(x_ref, ef_ref, wp_ref, w1_ref, b1_ref, w2_ref, b2_ref,
                out_ref):
    h = jnp.dot(ef_ref[...], w1_ref[...],
                preferred_element_type=jnp.float32) + b1_ref[...]
    h = h * lax.logistic(h)
    wp = wp_ref[...]
    wh = jnp.dot(w2_ref[...], wp[CAT_DIM:], preferred_element_type=jnp.float32)
    bias = jnp.dot(b2_ref[...], wp[CAT_DIM:],
                   preferred_element_type=jnp.float32)
    y = jnp.dot(x_ref[...][:, :CAT_DIM], wp[:CAT_DIM],
                preferred_element_type=jnp.float32)
    y = y + jnp.dot(h, wh, preferred_element_type=jnp.float32) + bias
    out_ref[...] = y * lax.logistic(y)


@functools.lru_cache(maxsize=None)
def _make_dense(n):
    return pl.pallas_call(
        _dense_body,
        grid=(-(-n // BLK),),
        in_specs=[
            pl.BlockSpec((BLK, OUT_DIM), lambda b: (b, 0)),
            pl.BlockSpec((BLK, CONT_IN), lambda b: (b, 0)),
            pl.BlockSpec((TOTAL_DIM, OUT_DIM), lambda b: (0, 0)),
            pl.BlockSpec((CONT_IN, EMB_CONT), lambda b: (0, 0)),
            pl.BlockSpec((1, EMB_CONT), lambda b: (0, 0)),
            pl.BlockSpec((EMB_CONT, EMB_CONT), lambda b: (0, 0)),
            pl.BlockSpec((1, EMB_CONT), lambda b: (0, 0)),
        ],
        out_specs=pl.BlockSpec((BLK, OUT_DIM), lambda b: (b, 0)),
        out_shape=jax.ShapeDtypeStruct((n, OUT_DIM), jnp.float32),
    )


def kernel(batch, atomic_type, total_charge, external_field,
           emb_atomic, emb_charge, W1, b1, W2, b2, Wp):
    n = batch.shape[0]
    pad = NPAD - n
    at = jnp.pad(atomic_type.astype(jnp.int32).reshape(-1), (0, pad))
    bt = jnp.pad(batch.astype(jnp.int32), (0, pad))

    gg = _prep(total_charge.astype(jnp.int32).reshape(-1, 1), emb_charge)
    x = _make_gather()(at.reshape(NW * STRIPS, SL),
                       bt.reshape(NW * STRIPS, SL), emb_atomic, gg)
    return _make_dense(n)(x, external_field, Wp, W1, b1.reshape(1, -1),
                          W2, b2.reshape(1, -1))
